# baseline (speedup 1.0000x reference)
import jax
import jax.numpy as jnp
from jax.experimental import pallas as pl
from jax.experimental.pallas import tpu as pltpu

_N = 4096
_D = 128
_BM = 2048
_BN = 2048
_SUB = 512
_NSUB = _BM // _SUB
_GRID = _N // _BM
_NT = _N // _SUB


def _seg_body(interact_ref, krow_ref, kcol_ref, z1_ref, z2_ref, out_ref):
    bi = pl.program_id(0)
    bj = pl.program_id(1)

    for si in range(_NSUB):
        for sj in range(_NSUB):
            gi = bi * _NSUB + si
            gj = bj * _NSUB + sj
            inter = interact_ref[gi, gj] != 0
            rs = slice(si * _SUB, (si + 1) * _SUB)
            cs = slice(sj * _SUB, (sj + 1) * _SUB)

            def _masked_prod(si=si, sj=sj, gj=gj):
                a = z1_ref[si * _SUB:(si + 1) * _SUB, :]
                b = z2_ref[pl.ds(gj * _SUB, _SUB), :]
                prod = jax.lax.dot_general(
                    a, b, (((1,), (1,)), ((), ())),
                    preferred_element_type=jnp.float32)
                rk = krow_ref[si * _SUB:(si + 1) * _SUB, :]
                ck = kcol_ref[:, sj * _SUB:(sj + 1) * _SUB]
                return prod, rk == ck

            @pl.when(inter & (gi == gj))
            def _compute_diag(rs=rs, cs=cs, mp=_masked_prod):
                prod, mask = mp()
                rid = jax.lax.broadcasted_iota(jnp.int32, (_SUB, _SUB), 0)
                cid = jax.lax.broadcasted_iota(jnp.int32, (_SUB, _SUB), 1)
                mask = mask & (rid != cid)
                out_ref[rs, cs] = jnp.where(mask, prod, jnp.float32(0.0))

            @pl.when(inter & (gi != gj))
            def _compute_offdiag(rs=rs, cs=cs, mp=_masked_prod):
                prod, mask = mp()
                out_ref[rs, cs] = jnp.where(mask, prod, jnp.float32(0.0))

            @pl.when(jnp.logical_not(inter))
            def _zero(rs=rs, cs=cs):
                out_ref[rs, cs] = jnp.zeros((_SUB, _SUB), jnp.float32)


def kernel(z1, z2, cls_label, batch):
    cls = cls_label.astype(jnp.int32)
    bat = batch.astype(jnp.int32)
    n = cls.shape[0]

    valid = (cls != 24) & (cls != 25) & (cls != 26)
    key = jnp.where(valid, bat * 32 + cls, -jnp.arange(n, dtype=jnp.int32) - 1)
    krow = key.reshape(n, 1)
    kcol = key.reshape(1, n)

    tb = bat.reshape(_NT, _SUB)
    bmin = tb[:, 0]
    bmax = tb[:, -1]
    interact = ((bmin[:, None] <= bmax[None, :])
                & (bmin[None, :] <= bmax[:, None])).astype(jnp.int32)

    out = pl.pallas_call(
        _seg_body,
        grid=(_GRID, _GRID),
        in_specs=[
            pl.BlockSpec(memory_space=pltpu.SMEM),
            pl.BlockSpec((_BM, 1), lambda i, j: (i, 0)),
            pl.BlockSpec((1, _BN), lambda i, j: (0, j)),
            pl.BlockSpec((_BM, _D), lambda i, j: (i, 0)),
            pl.BlockSpec((_N, _D), lambda i, j: (0, 0)),
        ],
        out_specs=pl.BlockSpec((_BM, _BN), lambda i, j: (i, j)),
        out_shape=jax.ShapeDtypeStruct((n, n), jnp.float32),
        compiler_params=pltpu.CompilerParams(
            dimension_semantics=("parallel", "parallel")),
    )(interact, krow, kcol, z1, z2)
    return out


# speedup vs baseline: 1.0350x; 1.0350x over previous
import jax
import jax.numpy as jnp
from jax.experimental import pallas as pl
from jax.experimental.pallas import tpu as pltpu

_N = 4096
_D = 128
_BM = 1024
_BN = 4096
_SUB = 512
_NSI = _BM // _SUB
_NSJ = _BN // _SUB
_NT = _N // _SUB


def _seg_body(interact_ref, krow_ref, kcol_ref, z1_ref, z2_ref, out_ref):
    bi = pl.program_id(0)

    for si in range(_NSI):
        for gj in range(_NSJ):
            gi = bi * _NSI + si
            inter = interact_ref[gi, gj] != 0
            rs = slice(si * _SUB, (si + 1) * _SUB)
            cs = slice(gj * _SUB, (gj + 1) * _SUB)

            def _masked_prod(si=si, gj=gj):
                a = z1_ref[si * _SUB:(si + 1) * _SUB, :]
                b = z2_ref[gj * _SUB:(gj + 1) * _SUB, :]
                prod = jax.lax.dot_general(
                    a, b, (((1,), (1,)), ((), ())),
                    preferred_element_type=jnp.float32)
                rk = krow_ref[si * _SUB:(si + 1) * _SUB, :]
                ck = kcol_ref[:, gj * _SUB:(gj + 1) * _SUB]
                return prod, rk == ck

            @pl.when(inter & (gi == gj))
            def _compute_diag(rs=rs, cs=cs, mp=_masked_prod):
                prod, mask = mp()
                rid = jax.lax.broadcasted_iota(jnp.int32, (_SUB, _SUB), 0)
                cid = jax.lax.broadcasted_iota(jnp.int32, (_SUB, _SUB), 1)
                mask = mask & (rid != cid)
                out_ref[rs, cs] = jnp.where(mask, prod, jnp.float32(0.0))

            @pl.when(inter & (gi != gj))
            def _compute_offdiag(rs=rs, cs=cs, mp=_masked_prod):
                prod, mask = mp()
                out_ref[rs, cs] = jnp.where(mask, prod, jnp.float32(0.0))

            @pl.when(jnp.logical_not(inter))
            def _zero(rs=rs, cs=cs):
                out_ref[rs, cs] = jnp.zeros((_SUB, _SUB), jnp.float32)


def kernel(z1, z2, cls_label, batch):
    cls = cls_label.astype(jnp.int32)
    bat = batch.astype(jnp.int32)
    n = cls.shape[0]

    valid = (cls != 24) & (cls != 25) & (cls != 26)
    key = jnp.where(valid, bat * 32 + cls, -jnp.arange(n, dtype=jnp.int32) - 1)
    krow = key.reshape(n, 1)
    kcol = key.reshape(1, n)

    tb = bat.reshape(_NT, _SUB)
    bmin = tb[:, 0]
    bmax = tb[:, -1]
    interact = ((bmin[:, None] <= bmax[None, :])
                & (bmin[None, :] <= bmax[:, None])).astype(jnp.int32)

    out = pl.pallas_call(
        _seg_body,
        grid=(_N // _BM,),
        in_specs=[
            pl.BlockSpec(memory_space=pltpu.SMEM),
            pl.BlockSpec((_BM, 1), lambda i: (i, 0)),
            pl.BlockSpec((1, _N), lambda i: (0, 0)),
            pl.BlockSpec((_BM, _D), lambda i: (i, 0)),
            pl.BlockSpec((_N, _D), lambda i: (0, 0)),
        ],
        out_specs=pl.BlockSpec((_BM, _BN), lambda i: (i, 0)),
        out_shape=jax.ShapeDtypeStruct((n, n), jnp.float32),
        compiler_params=pltpu.CompilerParams(
            dimension_semantics=("parallel",)),
    )(interact, krow, kcol, z1, z2)
    return out
